# Initial kernel scaffold; baseline (speedup 1.0000x reference)
import jax
import jax.numpy as jnp
from jax.experimental import pallas as pl


def kernel(x, coeffs, bias):
    raise NotImplementedError("write your pallas kernel here")


# speedup vs baseline: 394.9850x; 394.9850x over previous
import functools

import jax
import jax.numpy as jnp
import numpy as np
from jax import lax
from jax.experimental import pallas as pl
from jax.experimental.pallas import tpu as pltpu
from jax.experimental.pallas import tpu_sc as plsc

B = 16384
D = 768
P = D // 2
G = 16
NW = 32
PPW = P // NW
CPW = 2 * PPW
TW = PPW * G * G * 3
R = 2048
NCHUNK = B // R
NR16 = R // 16

CLMAX = float(np.float32(np.float32(1.0) - np.float32(1e-6)) * np.float32(16.0))

_mesh = plsc.VectorSubcoreMesh(core_axis_name="c", subcore_axis_name="s")


@functools.partial(
    pl.kernel,
    mesh=_mesh,
    compiler_params=pltpu.CompilerParams(needs_layout_passes=False),
    out_type=jax.ShapeDtypeStruct((NW, B), jnp.float32),
    scratch_types=[
        pltpu.VMEM((TW,), jnp.float32),
        pltpu.VMEM((CPW, R), jnp.float32),
        pltpu.VMEM((CPW, R), jnp.float32),
        pltpu.VMEM((1, R), jnp.float32),
        pltpu.SemaphoreType.DMA,
        pltpu.SemaphoreType.DMA,
    ],
)
def _spline_partials(xt_hbm, ctab_hbm, out_hbm, tab_v, xb0, xb1, ob_v, sem0, sem1):
    wid = lax.axis_index("s") * 2 + lax.axis_index("c")
    row0 = wid * CPW

    pltpu.sync_copy(ctab_hbm.at[pl.ds(wid * TW, TW)], tab_v)

    bufs = (xb0, xb1)
    sems = (sem0, sem1)
    copies = [None, None]
    copies[0] = pltpu.async_copy(
        xt_hbm.at[pl.ds(row0, CPW), pl.ds(0, R)], xb0, sem0)

    for c in range(NCHUNK):
        s = c % 2
        if c + 1 < NCHUNK:
            copies[1 - s] = pltpu.async_copy(
                xt_hbm.at[pl.ds(row0, CPW), pl.ds((c + 1) * R, R)],
                bufs[1 - s], sems[1 - s])
        copies[s].wait()
        buf = bufs[s]

        def r16_body(i, carry):
            acc = jnp.zeros((16,), jnp.float32)
            for dp in range(PPW):
                a = buf[2 * dp, pl.ds(i * 16, 16)]
                b = buf[2 * dp + 1, pl.ds(i * 16, 16)]
                fa = jnp.minimum(jnp.maximum(a * 8.0 + 8.0, 0.0), CLMAX)
                fb = jnp.minimum(jnp.maximum(b * 8.0 + 8.0, 0.0), CLMAX)
                ia = fa.astype(jnp.int32)
                ib = fb.astype(jnp.int32)
                cell = ia * 48 + ib * 3 + (dp * G * G * 3)
                c0 = plsc.load_gather(tab_v, [cell])
                c1 = plsc.load_gather(tab_v, [cell + 1])
                c2 = plsc.load_gather(tab_v, [cell + 2])
                acc = acc + (c0 + c1 * fa + c2 * fb)
            ob_v[0, pl.ds(i * 16, 16)] = acc
            return carry

        lax.fori_loop(0, NR16, r16_body, 0)
        pltpu.sync_copy(ob_v, out_hbm.at[pl.ds(wid, 1), pl.ds(c * R, R)])


def _reduce_body(p_ref, b_ref, o_ref):
    o_ref[...] = jnp.sum(p_ref[...], axis=0, keepdims=True) + b_ref[...]


def kernel(x, coeffs, bias):
    scale = jnp.array([1.0, 0.0625, 0.0625], jnp.float32)
    ctab = (coeffs.reshape(P, G * G, 3) * scale).reshape(-1)
    partials = _spline_partials(x.T, ctab)
    out = pl.pallas_call(
        _reduce_body,
        out_shape=jax.ShapeDtypeStruct((1, B), jnp.float32),
    )(partials, bias.reshape(1, 1))
    return out.reshape(B, 1)
